# baseline (device time: 17383 ns/iter reference)
import jax
import jax.numpy as jnp
from jax import lax
from jax.experimental import pallas as pl
from jax.experimental.pallas import tpu as pltpu

Z = 4
X = 2

_ORDER = {k: sorted((p for p in range(Z) if p != k), key=lambda p: abs(p - k))
          for k in range(Z)}


def kernel(x):
    m, n = x.shape
    blk = n // Z
    half = m // X

    def body(x_ref, out_ref, zsend, zrecv, xsend, xrecv):
        mx = lax.axis_index("x")
        my = lax.axis_index("y")
        mz = lax.axis_index("z")
        px = 1 - mx

        barrier_sem = pltpu.get_barrier_semaphore()
        for r in range(1, Z):
            pl.semaphore_signal(
                barrier_sem, inc=1,
                device_id=(mx, my, (mz + r) % Z),
                device_id_type=pl.DeviceIdType.MESH,
            )
        pl.semaphore_signal(
            barrier_sem, inc=1,
            device_id=(px, my, mz),
            device_id_type=pl.DeviceIdType.MESH,
        )
        pl.semaphore_wait(barrier_sem, Z)

        zs = []
        for r in range(1, Z):
            q = (mz + r) % Z
            rdma = pltpu.make_async_remote_copy(
                src_ref=x_ref.at[pl.ds(mx * half, half), pl.ds(q * blk, blk)],
                dst_ref=out_ref.at[pl.ds(mz * m + mx * half, half), :],
                send_sem=zsend.at[r - 1],
                recv_sem=zrecv.at[r - 1],
                device_id=(mx, my, q),
                device_id_type=pl.DeviceIdType.MESH,
            )
            rdma.start()
            zs.append(rdma)

        out_ref[pl.ds(mz * m, m), :] = x_ref[:, pl.ds(mz * blk, blk)]

        for k in range(Z):
            @pl.when(mz == k)
            def _(k=k):
                xds = []
                for p in _ORDER[k]:
                    s = (k - p) % Z - 1
                    rows = pl.ds(p * m + mx * half, half)
                    zin = pltpu.make_async_remote_copy(
                        src_ref=out_ref.at[rows, :],
                        dst_ref=out_ref.at[rows, :],
                        send_sem=zsend.at[s],
                        recv_sem=zrecv.at[s],
                        device_id=(mx, my, p),
                        device_id_type=pl.DeviceIdType.MESH,
                    )
                    zin.wait_recv()
                    fwd = pltpu.make_async_remote_copy(
                        src_ref=out_ref.at[rows, :],
                        dst_ref=out_ref.at[rows, :],
                        send_sem=xsend.at[s],
                        recv_sem=xrecv.at[s],
                        device_id=(px, my, k),
                        device_id_type=pl.DeviceIdType.MESH,
                    )
                    fwd.start()
                    xds.append(fwd)
                for fwd in xds:
                    fwd.wait()

        for r in range(1, Z):
            zs[r - 1].wait_send()

    out_shape = jax.ShapeDtypeStruct((Z * m, blk), x.dtype)
    return pl.pallas_call(
        body,
        out_shape=out_shape,
        in_specs=[pl.BlockSpec(memory_space=pltpu.VMEM)],
        out_specs=pl.BlockSpec(memory_space=pltpu.VMEM),
        scratch_shapes=[
            pltpu.SemaphoreType.DMA((Z - 1,)),
            pltpu.SemaphoreType.DMA((Z - 1,)),
            pltpu.SemaphoreType.DMA((Z - 1,)),
            pltpu.SemaphoreType.DMA((Z - 1,)),
        ],
        compiler_params=pltpu.CompilerParams(collective_id=0),
    )(x)


# device time: 16136 ns/iter; 1.0773x vs baseline; 1.0773x over previous
import jax
import jax.numpy as jnp
from jax import lax
from jax.experimental import pallas as pl
from jax.experimental.pallas import tpu as pltpu

Z = 4


def kernel(x):
    m, n = x.shape
    blk = n // Z

    def body(x_ref, out_ref, send_sems, recv_sems, local_sem):
        my_x = lax.axis_index("x")
        my_y = lax.axis_index("y")
        my_z = lax.axis_index("z")

        barrier_sem = pltpu.get_barrier_semaphore()
        for r in range(1, Z):
            pl.semaphore_signal(
                barrier_sem, inc=1,
                device_id=(my_x, my_y, (my_z + r) % Z),
                device_id_type=pl.DeviceIdType.MESH,
            )
        pl.semaphore_wait(barrier_sem, Z - 1)

        diag = pltpu.make_async_copy(
            x_ref.at[:, pl.ds(my_z * blk, blk)],
            out_ref.at[pl.ds(my_z * m, m), :],
            local_sem,
        )
        diag.start()

        rdmas = []
        for r in range(1, Z):
            tgt = (my_z + r) % Z
            rdma = pltpu.make_async_remote_copy(
                src_ref=x_ref.at[:, pl.ds(tgt * blk, blk)],
                dst_ref=out_ref.at[pl.ds(my_z * m, m), :],
                send_sem=send_sems.at[r - 1],
                recv_sem=recv_sems.at[r - 1],
                device_id=(my_x, my_y, tgt),
                device_id_type=pl.DeviceIdType.MESH,
            )
            rdma.start()
            rdmas.append(rdma)

        diag.wait()
        for rdma in rdmas:
            rdma.wait()

    out_shape = jax.ShapeDtypeStruct((Z * m, blk), x.dtype)
    return pl.pallas_call(
        body,
        out_shape=out_shape,
        in_specs=[pl.BlockSpec(memory_space=pltpu.HBM)],
        out_specs=pl.BlockSpec(memory_space=pltpu.HBM),
        scratch_shapes=[
            pltpu.SemaphoreType.DMA((Z - 1,)),
            pltpu.SemaphoreType.DMA((Z - 1,)),
            pltpu.SemaphoreType.DMA,
        ],
        compiler_params=pltpu.CompilerParams(collective_id=0),
    )(x)
